# Initial kernel scaffold; baseline (speedup 1.0000x reference)
import jax
import jax.numpy as jnp
from jax.experimental import pallas as pl


def kernel(x, w_embed, b_embed, w_inter, b_inter):
    raise NotImplementedError("write your pallas kernel here")


# speedup vs baseline: 4.3724x; 4.3724x over previous
import jax
import jax.numpy as jnp
from jax import lax
from jax.experimental import pallas as pl
from jax.experimental.pallas import tpu as pltpu

_F32 = jnp.float32
_PREC = lax.Precision.HIGHEST


def _proj_kernel(x_ref, wet_ref, be_ref, wwt_ref, bw_ref,
                 emb_ref, w_ref):
    emb_ref[...] = jnp.dot(x_ref[...], wet_ref[...], preferred_element_type=_F32,
                           precision=_PREC) + be_ref[...]
    w_ref[...] = jnp.dot(x_ref[...], wwt_ref[...], preferred_element_type=_F32,
                         precision=_PREC) + bw_ref[...]


def _interact_kernel(emb_ref, embt_ref, w_ref, o_ref, *, tb, e_dim, ci):
    for i0 in range(0, e_dim, ci):
        wc = w_ref[i0:i0 + ci, :]
        eic = embt_ref[0, i0:i0 + ci, :]
        for b in range(tb):
            ej = emb_ref[b:b + 1, :]
            o_ref[b, i0:i0 + ci, :] = eic[:, b:b + 1] * (ej * wc)


def _project(x, we_t, be, ww_t, bw):
    B, D = x.shape
    E = we_t.shape[1]
    nc = 2 if E % 256 == 0 else 1
    ec = E // nc
    cparams = pltpu.CompilerParams(
        dimension_semantics=("parallel",),
        vmem_limit_bytes=int(min(48 << 20,
                                 (B * D + 2 * (2 * D * ec + 2 * ec)
                                  + 6 * B * ec) * 4 + (4 << 20))))
    return pl.pallas_call(
        _proj_kernel,
        out_shape=(jax.ShapeDtypeStruct((B, E), _F32),
                   jax.ShapeDtypeStruct((B, E), _F32)),
        grid=(nc,),
        in_specs=[
            pl.BlockSpec((B, D), lambda c: (0, 0)),
            pl.BlockSpec((D, ec), lambda c: (0, c)),
            pl.BlockSpec((1, ec), lambda c: (0, c)),
            pl.BlockSpec((D, ec), lambda c: (0, c)),
            pl.BlockSpec((1, ec), lambda c: (0, c)),
        ],
        out_specs=(pl.BlockSpec((B, ec), lambda c: (0, c)),
                   pl.BlockSpec((B, ec), lambda c: (0, c))),
        compiler_params=cparams,
    )(x, we_t, be, ww_t, bw)


def _interact(emb, embt, w):
    B, E = emb.shape
    tb = embt.shape[2]
    nb = B // tb
    ci = 128 if E % 128 == 0 else E
    out_block = tb * E * E * 4
    cparams = pltpu.CompilerParams(
        dimension_semantics=("parallel",),
        vmem_limit_bytes=int(min(60 << 20, 2 * out_block + (8 << 20))))
    return pl.pallas_call(
        lambda er, etr, wr, orf: _interact_kernel(er, etr, wr, orf,
                                                  tb=tb, e_dim=E, ci=ci),
        out_shape=jax.ShapeDtypeStruct((B, E, E), _F32),
        grid=(nb,),
        in_specs=[
            pl.BlockSpec((tb, E), lambda b: (b, 0)),
            pl.BlockSpec((1, E, tb), lambda b: (b, 0, 0)),
            pl.BlockSpec((E, E), lambda b: (0, 0)),
        ],
        out_specs=pl.BlockSpec((tb, E, E), lambda b: (b, 0, 0)),
        compiler_params=cparams,
    )(emb, embt, w)


def kernel(x, w_embed, b_embed, w_inter, b_inter):
    B, D = x.shape
    E = w_embed.shape[0]
    assert B == E, "interaction mechanism requires batch_size == embed_dim"
    x = x.astype(_F32)
    we_t = jnp.transpose(w_embed.astype(_F32))
    ww_t = jnp.transpose(w_inter.astype(_F32))
    be = b_embed.astype(_F32).reshape(1, E)
    bw = b_inter.astype(_F32).reshape(1, E)
    emb, w = _project(x, we_t, be, ww_t, bw)
    tb = 8 if B % 8 == 0 else B
    embt = emb.reshape(B // tb, tb, E).transpose(0, 2, 1)
    return _interact(emb, embt, w)


# speedup vs baseline: 4.5908x; 1.0500x over previous
import jax
import jax.numpy as jnp
from jax import lax
from jax.experimental import pallas as pl
from jax.experimental.pallas import tpu as pltpu

_F32 = jnp.float32
_PREC = lax.Precision.HIGHEST


_DN_TRANS_B = (((1,), (1,)), ((), ()))


def _proj_kernel(x_ref, we_ref, be_ref, ww_ref, bw_ref,
                 emb_ref, w_ref):
    x = x_ref[...]
    emb_ref[...] = lax.dot_general(x, we_ref[...], _DN_TRANS_B,
                                   preferred_element_type=_F32,
                                   precision=_PREC) + be_ref[...]
    w_ref[...] = lax.dot_general(x, ww_ref[...], _DN_TRANS_B,
                                 preferred_element_type=_F32,
                                 precision=_PREC) + bw_ref[...]


def _interact_kernel(emb_ref, embt_ref, w_ref, o_ref, *, tb, e_dim, ci):
    for i0 in range(0, e_dim, ci):
        wc = w_ref[i0:i0 + ci, :]
        eic = embt_ref[0, i0:i0 + ci, :]
        for b in range(tb):
            ej = emb_ref[b:b + 1, :]
            o_ref[b, i0:i0 + ci, :] = eic[:, b:b + 1] * (ej * wc)


def _project(x, we, be, ww, bw):
    B, D = x.shape
    E = we.shape[0]
    nc = 2 if E % 256 == 0 else 1
    ec = E // nc
    cparams = pltpu.CompilerParams(
        dimension_semantics=("parallel",),
        vmem_limit_bytes=56 << 20)
    return pl.pallas_call(
        _proj_kernel,
        out_shape=(jax.ShapeDtypeStruct((B, E), _F32),
                   jax.ShapeDtypeStruct((B, E), _F32)),
        grid=(nc,),
        in_specs=[
            pl.BlockSpec((B, D), lambda c: (0, 0)),
            pl.BlockSpec((ec, D), lambda c: (c, 0)),
            pl.BlockSpec((1, ec), lambda c: (0, c)),
            pl.BlockSpec((ec, D), lambda c: (c, 0)),
            pl.BlockSpec((1, ec), lambda c: (0, c)),
        ],
        out_specs=(pl.BlockSpec((B, ec), lambda c: (0, c)),
                   pl.BlockSpec((B, ec), lambda c: (0, c))),
        compiler_params=cparams,
    )(x, we, be, ww, bw)


def _interact(emb, embt, w):
    B, E = emb.shape
    tb = embt.shape[2]
    nb = B // tb
    ci = 128 if E % 128 == 0 else E
    out_block = tb * E * E * 4
    cparams = pltpu.CompilerParams(
        dimension_semantics=("parallel",),
        vmem_limit_bytes=int(min(60 << 20, 2 * out_block + (8 << 20))))
    return pl.pallas_call(
        lambda er, etr, wr, orf: _interact_kernel(er, etr, wr, orf,
                                                  tb=tb, e_dim=E, ci=ci),
        out_shape=jax.ShapeDtypeStruct((B, E, E), _F32),
        grid=(nb,),
        in_specs=[
            pl.BlockSpec((tb, E), lambda b: (b, 0)),
            pl.BlockSpec((1, E, tb), lambda b: (b, 0, 0)),
            pl.BlockSpec((E, E), lambda b: (0, 0)),
        ],
        out_specs=pl.BlockSpec((tb, E, E), lambda b: (b, 0, 0)),
        compiler_params=cparams,
    )(emb, embt, w)


def kernel(x, w_embed, b_embed, w_inter, b_inter):
    B, D = x.shape
    E = w_embed.shape[0]
    assert B == E, "interaction mechanism requires batch_size == embed_dim"
    x = x.astype(_F32)
    be = b_embed.astype(_F32).reshape(1, E)
    bw = b_inter.astype(_F32).reshape(1, E)
    emb, w = _project(x, w_embed.astype(_F32), be, w_inter.astype(_F32), bw)
    tb = 8 if B % 8 == 0 else B
    embt = emb.reshape(B // tb, tb, E).transpose(0, 2, 1)
    return _interact(emb, embt, w)


# speedup vs baseline: 4.8527x; 1.0571x over previous
import jax
import jax.numpy as jnp
from jax import lax
from jax.experimental import pallas as pl
from jax.experimental.pallas import tpu as pltpu

_F32 = jnp.float32
_PREC = lax.Precision.DEFAULT


_DN_TRANS_B = (((1,), (1,)), ((), ()))


def _proj_kernel(x_ref, we_ref, be_ref, ww_ref, bw_ref,
                 emb_ref, w_ref):
    x = x_ref[...]
    emb_ref[...] = lax.dot_general(x, we_ref[...], _DN_TRANS_B,
                                   preferred_element_type=_F32,
                                   precision=_PREC) + be_ref[...]
    w_ref[...] = lax.dot_general(x, ww_ref[...], _DN_TRANS_B,
                                 preferred_element_type=_F32,
                                 precision=_PREC) + bw_ref[...]


def _interact_kernel(emb_ref, embt_ref, w_ref, o_ref, *, tb, e_dim, ci):
    for i0 in range(0, e_dim, ci):
        wc = w_ref[i0:i0 + ci, :]
        eic = embt_ref[0, i0:i0 + ci, :]
        for b in range(tb):
            ej = emb_ref[b:b + 1, :]
            o_ref[b, i0:i0 + ci, :] = eic[:, b:b + 1] * (ej * wc)


def _project(x, we, be, ww, bw):
    B, D = x.shape
    E = we.shape[0]
    nc = 2 if E % 256 == 0 else 1
    ec = E // nc
    cparams = pltpu.CompilerParams(
        dimension_semantics=("parallel",),
        vmem_limit_bytes=56 << 20)
    return pl.pallas_call(
        _proj_kernel,
        out_shape=(jax.ShapeDtypeStruct((B, E), _F32),
                   jax.ShapeDtypeStruct((B, E), _F32)),
        grid=(nc,),
        in_specs=[
            pl.BlockSpec((B, D), lambda c: (0, 0)),
            pl.BlockSpec((ec, D), lambda c: (c, 0)),
            pl.BlockSpec((1, ec), lambda c: (0, c)),
            pl.BlockSpec((ec, D), lambda c: (c, 0)),
            pl.BlockSpec((1, ec), lambda c: (0, c)),
        ],
        out_specs=(pl.BlockSpec((B, ec), lambda c: (0, c)),
                   pl.BlockSpec((B, ec), lambda c: (0, c))),
        compiler_params=cparams,
    )(x, we, be, ww, bw)


def _interact(emb, embt, w):
    B, E = emb.shape
    tb = embt.shape[2]
    nb = B // tb
    ci = 128 if E % 128 == 0 else E
    out_block = tb * E * E * 4
    cparams = pltpu.CompilerParams(
        dimension_semantics=("parallel",),
        vmem_limit_bytes=int(min(60 << 20, 2 * out_block + (8 << 20))))
    return pl.pallas_call(
        lambda er, etr, wr, orf: _interact_kernel(er, etr, wr, orf,
                                                  tb=tb, e_dim=E, ci=ci),
        out_shape=jax.ShapeDtypeStruct((B, E, E), _F32),
        grid=(nb,),
        in_specs=[
            pl.BlockSpec((tb, E), lambda b: (b, 0)),
            pl.BlockSpec((1, E, tb), lambda b: (b, 0, 0)),
            pl.BlockSpec((E, E), lambda b: (0, 0)),
        ],
        out_specs=pl.BlockSpec((tb, E, E), lambda b: (b, 0, 0)),
        compiler_params=cparams,
    )(emb, embt, w)


def kernel(x, w_embed, b_embed, w_inter, b_inter):
    B, D = x.shape
    E = w_embed.shape[0]
    assert B == E, "interaction mechanism requires batch_size == embed_dim"
    x = x.astype(_F32)
    be = b_embed.astype(_F32).reshape(1, E)
    bw = b_inter.astype(_F32).reshape(1, E)
    emb, w = _project(x, w_embed.astype(_F32), be, w_inter.astype(_F32), bw)
    tb = 8 if B % 8 == 0 else B
    embt = emb.reshape(B // tb, tb, E).transpose(0, 2, 1)
    return _interact(emb, embt, w)


# speedup vs baseline: 5.0509x; 1.0408x over previous
import jax
import jax.numpy as jnp
from jax import lax
from jax.experimental import pallas as pl
from jax.experimental.pallas import tpu as pltpu

_F32 = jnp.float32
_PREC = lax.Precision.DEFAULT


_DN_TRANS_B = (((1,), (1,)), ((), ()))


def _proj_kernel(x_ref, we_ref, be_ref, ww_ref, bw_ref,
                 emb_ref, w_ref):
    x = x_ref[...]
    emb_ref[...] = lax.dot_general(x, we_ref[...], _DN_TRANS_B,
                                   preferred_element_type=_F32,
                                   precision=_PREC) + be_ref[...]
    w_ref[...] = lax.dot_general(x, ww_ref[...], _DN_TRANS_B,
                                 preferred_element_type=_F32,
                                 precision=_PREC) + bw_ref[...]


def _interact_kernel(emb_ref, w_ref, o_ref, *, tb, e_dim, ci):
    for i0 in range(0, e_dim, ci):
        wc = w_ref[i0:i0 + ci, :]
        for b in range(tb):
            ej = emb_ref[b:b + 1, :]
            ei = jnp.transpose(emb_ref[b:b + 1, i0:i0 + ci])
            o_ref[b, i0:i0 + ci, :] = ei * (ej * wc)


def _project(x, we, be, ww, bw):
    B, D = x.shape
    E = we.shape[0]
    nc = 2 if E % 256 == 0 else 1
    ec = E // nc
    cparams = pltpu.CompilerParams(
        dimension_semantics=("parallel",),
        vmem_limit_bytes=56 << 20)
    return pl.pallas_call(
        _proj_kernel,
        out_shape=(jax.ShapeDtypeStruct((B, E), _F32),
                   jax.ShapeDtypeStruct((B, E), _F32)),
        grid=(nc,),
        in_specs=[
            pl.BlockSpec((B, D), lambda c: (0, 0)),
            pl.BlockSpec((ec, D), lambda c: (c, 0)),
            pl.BlockSpec((1, ec), lambda c: (0, c)),
            pl.BlockSpec((ec, D), lambda c: (c, 0)),
            pl.BlockSpec((1, ec), lambda c: (0, c)),
        ],
        out_specs=(pl.BlockSpec((B, ec), lambda c: (0, c)),
                   pl.BlockSpec((B, ec), lambda c: (0, c))),
        compiler_params=cparams,
    )(x, we, be, ww, bw)


def _interact(emb, w):
    B, E = emb.shape
    tb = 8 if B % 8 == 0 else B
    nb = B // tb
    ci = 128 if E % 128 == 0 else E
    out_block = tb * E * E * 4
    cparams = pltpu.CompilerParams(
        dimension_semantics=("parallel",),
        vmem_limit_bytes=int(min(60 << 20, 2 * out_block + (8 << 20))))
    return pl.pallas_call(
        lambda er, wr, orf: _interact_kernel(er, wr, orf,
                                             tb=tb, e_dim=E, ci=ci),
        out_shape=jax.ShapeDtypeStruct((B, E, E), _F32),
        grid=(nb,),
        in_specs=[
            pl.BlockSpec((tb, E), lambda b: (b, 0)),
            pl.BlockSpec((E, E), lambda b: (0, 0)),
        ],
        out_specs=pl.BlockSpec((tb, E, E), lambda b: (b, 0, 0)),
        compiler_params=cparams,
    )(emb, w)


def kernel(x, w_embed, b_embed, w_inter, b_inter):
    B, D = x.shape
    E = w_embed.shape[0]
    assert B == E, "interaction mechanism requires batch_size == embed_dim"
    x = x.astype(_F32)
    be = b_embed.astype(_F32).reshape(1, E)
    bw = b_inter.astype(_F32).reshape(1, E)
    emb, w = _project(x, w_embed.astype(_F32), be, w_inter.astype(_F32), bw)
    return _interact(emb, w)
